# baseline (speedup 1.0000x reference)
import functools

import jax
import jax.numpy as jnp
from jax import lax
from jax.experimental import pallas as pl
from jax.experimental.pallas import tpu as pltpu
from jax.experimental.pallas import tpu_sc as plsc

N_NODES = 10000
N_FEATS = 128
N_EDGES = 320000

NC = 2
NS = 16
NW = NC * NS
EDGES_PER_TILE = N_EDGES // NW
CHUNK = 50
CHUNKS_PER_TILE = EDGES_PER_TILE // CHUNK
IDXG = 40
N_GROUPS = CHUNKS_PER_TILE // IDXG
INNER = 8
NBUF = 4
LOOKAHEAD = 3
LANES = 16
CP_ROWS = 200
CP_CHUNKS = N_NODES // CP_ROWS
CP_ROUNDS = -(-CP_CHUNKS // NS)

_MESH = plsc.VectorSubcoreMesh(
    core_axis_name="c", subcore_axis_name="s", num_cores=NC, num_subcores=NS)
_SC_PARAMS = pltpu.CompilerParams(needs_layout_passes=False)


HIST_ROWS = 80


def _degrees_body(edges_hbm, out_src, out_dst,
                  idx_v, hist2d, idn_v, hsh_src, hsh_dst):
    c = lax.axis_index("c")
    s = lax.axis_index("s")
    wid = c * NS + s
    base = wid * CHUNKS_PER_TILE
    ones = jnp.full((LANES,), 1.0, jnp.float32)
    zeros = jnp.zeros((LANES,), jnp.float32)

    def zero_hist(i, _):
        for k in range(N_FEATS // LANES):
            hist2d[i, pl.ds(k * LANES, LANES)] = zeros
        return 0

    def fill_idn(i, _):
        idn_v[pl.ds(i * LANES, LANES)] = lax.iota(jnp.int32, LANES) + i * LANES
        return 0

    lax.fori_loop(0, HIST_ROWS, zero_hist, 0)
    lax.fori_loop(0, HIST_ROWS // LANES, fill_idn, 0)

    @pl.when(s == 0)
    def _():
        pltpu.sync_copy(hist2d, hsh_src)
        pltpu.sync_copy(hist2d, hsh_dst)

    plsc.subcore_barrier()

    tail_mask = lax.iota(jnp.int32, LANES) >= (LANES - CHUNK % LANES)

    def one_endpoint(ep, hist_sh, out_hbm, rezero):
        pltpu.sync_copy(edges_hbm.at[ep, pl.ds(base, CHUNKS_PER_TILE)], idx_v)

        def add_step(i, _):
            for k in range(CHUNK // LANES):
                idx = idx_v[i, pl.ds(k * LANES, LANES)]
                plsc.addupdate_scatter(hist2d, [idx >> 7, idx & 127], ones)
            idx = idx_v[i, pl.ds(CHUNK - LANES, LANES)]
            plsc.addupdate_scatter(hist2d, [idx >> 7, idx & 127], ones,
                                   mask=tail_mask)
            return 0

        lax.fori_loop(0, CHUNKS_PER_TILE, add_step, 0)
        pltpu.sync_copy(hist2d, hist_sh.at[idn_v], add=True)
        if rezero:
            lax.fori_loop(0, HIST_ROWS, zero_hist, 0)
        plsc.subcore_barrier()

        @pl.when(s == 0)
        def _():
            pltpu.sync_copy(hist_sh, out_hbm.at[c])

    one_endpoint(0, hsh_src, out_src, True)
    one_endpoint(1, hsh_dst, out_dst, False)


def _degrees(e3d):
    return pl.kernel(
        _degrees_body,
        out_type=(
            jax.ShapeDtypeStruct((NC, HIST_ROWS, N_FEATS), jnp.float32),
            jax.ShapeDtypeStruct((NC, HIST_ROWS, N_FEATS), jnp.float32),
        ),
        mesh=_MESH,
        scratch_types=[
            pltpu.VMEM((CHUNKS_PER_TILE, CHUNK), jnp.int32),
            pltpu.VMEM((HIST_ROWS, N_FEATS), jnp.float32),
            pltpu.VMEM((HIST_ROWS,), jnp.int32),
            pltpu.VMEM_SHARED((HIST_ROWS, N_FEATS), jnp.float32),
            pltpu.VMEM_SHARED((HIST_ROWS, N_FEATS), jnp.float32),
        ],
        compiler_params=_SC_PARAMS,
    )(e3d)


_MM_BLK = 1000


def _norms_body(hs_ref, hd_ref, ns_ref, nd_ref):
    def norm(h2):
        deg = (h2[0] + h2[1]).reshape(HIST_ROWS * N_FEATS)[:N_NODES]
        return lax.rsqrt(jnp.maximum(deg, 1.0))[:, None]

    ns_ref[...] = norm(hs_ref[...])
    nd_ref[...] = norm(hd_ref[...])


def _norms(hist_src, hist_dst):
    return pl.pallas_call(
        _norms_body,
        out_shape=(
            jax.ShapeDtypeStruct((N_NODES, 1), jnp.float32),
            jax.ShapeDtypeStruct((N_NODES, 1), jnp.float32),
        ),
    )(hist_src, hist_dst)


def _matmul_body(x_ref, n1_ref, w_ref, h_ref):
    h_ref[...] = jnp.dot(x_ref[...] * n1_ref[...], w_ref[...],
                         preferred_element_type=jnp.float32)


def _matmul(features, norm_src, weight):
    return pl.pallas_call(
        _matmul_body,
        grid=(N_NODES // _MM_BLK,),
        in_specs=[
            pl.BlockSpec((_MM_BLK, N_FEATS), lambda i: (i, 0)),
            pl.BlockSpec((_MM_BLK, 1), lambda i: (i, 0)),
            pl.BlockSpec((N_FEATS, N_FEATS), lambda i: (0, 0)),
        ],
        out_specs=pl.BlockSpec((_MM_BLK, N_FEATS), lambda i: (i, 0)),
        out_shape=jax.ShapeDtypeStruct((N_NODES, N_FEATS), jnp.float32),
    )(features, norm_src, weight)


def _message_body(h_hbm, e3d_hbm, zeros_hbm, out_hbm,
                  srcblk, dstblk, rows, acc_sh,
                  gsem0, gsem1, gsem2, gsem3, ssem0, ssem1, ssem2, ssem3,
                  isrc, idst):
    c = lax.axis_index("c")
    s = lax.axis_index("s")
    wid = c * NS + s

    for j in range(CP_ROUNDS):
        cid = s + NS * j

        @pl.when(cid < CP_CHUNKS)
        def _():
            pltpu.sync_copy(zeros_hbm, acc_sh.at[pl.ds(cid * CP_ROWS, CP_ROWS)])

    gsems = (gsem0, gsem1, gsem2, gsem3)
    ssems = (ssem0, ssem1, ssem2, ssem3)
    base = wid * CHUNKS_PER_TILE

    def gather(slot, j, b):
        return pltpu.async_copy(h_hbm.at[srcblk.at[slot, j]], rows.at[b],
                                gsems[b])

    def wait_gather(b):
        pltpu.make_async_copy(h_hbm.at[srcblk.at[0, 0]], rows.at[b],
                              gsems[b]).wait()

    def scatter(slot, j, b):
        return pltpu.async_copy(rows.at[b], acc_sh.at[dstblk.at[slot, j]],
                                ssems[b], add=True)

    def wait_scatter(b):
        pltpu.make_async_copy(rows.at[b], acc_sh.at[dstblk.at[0, 0]],
                              ssems[b]).wait()

    def load_idx(g1, slot, sem_pair):
        nbase = base + g1 * IDXG
        pltpu.async_copy(e3d_hbm.at[0, pl.ds(nbase, IDXG)],
                         srcblk.at[slot], sem_pair[0])
        pltpu.async_copy(e3d_hbm.at[1, pl.ds(nbase, IDXG)],
                         dstblk.at[slot], sem_pair[1])

    def wait_idx(slot):
        pltpu.make_async_copy(e3d_hbm.at[0, pl.ds(base, IDXG)],
                              srcblk.at[slot], isrc).wait()
        pltpu.make_async_copy(e3d_hbm.at[1, pl.ds(base, IDXG)],
                              dstblk.at[slot], idst).wait()

    pltpu.sync_copy(e3d_hbm.at[0, pl.ds(base, IDXG)], srcblk.at[0])
    pltpu.sync_copy(e3d_hbm.at[1, pl.ds(base, IDXG)], dstblk.at[0])
    for p in range(LOOKAHEAD):
        gather(0, p, p % NBUF)
    plsc.subcore_barrier()

    def group_step(g, _):
        slot = g % 2
        nslot = (g + 1) % 2

        def inner_step(q, _):
            for k in range(INNER):
                j = q * INNER + k
                jn = j + LOOKAHEAD
                b = k % NBUF
                bn = (k + LOOKAHEAD) % NBUF
                pb = (k - 1) % NBUF
                if k == 0:
                    @pl.when((g > 0) | (q > 0))
                    def _():
                        wait_scatter(pb)

                    @pl.when((q == 0) & (g + 1 < N_GROUPS))
                    def _():
                        load_idx(g + 1, nslot, (isrc, idst))
                else:
                    wait_scatter(pb)

                if k + LOOKAHEAD < INNER:
                    gather(slot, jn, bn)
                else:
                    @pl.when(jn < IDXG)
                    def _():
                        gather(slot, jn, bn)

                    @pl.when((jn >= IDXG) & (g + 1 < N_GROUPS))
                    def _():
                        if k == INNER - LOOKAHEAD:
                            wait_idx(nslot)
                        gather(nslot, jn - IDXG, bn)

                wait_gather(b)
                scatter(slot, j, b)
            return 0

        lax.fori_loop(0, IDXG // INNER, inner_step, 0)
        return 0

    lax.fori_loop(0, N_GROUPS, group_step, 0)
    wait_scatter((IDXG - 1) % NBUF)
    plsc.subcore_barrier()

    for j in range(CP_ROUNDS):
        cid = s + NS * j

        @pl.when(cid < CP_CHUNKS)
        def _():
            pltpu.sync_copy(acc_sh.at[pl.ds(cid * CP_ROWS, CP_ROWS)],
                            out_hbm.at[c, pl.ds(cid * CP_ROWS, CP_ROWS)])


def _message(h, e3d, zeros):
    return pl.kernel(
        _message_body,
        out_type=jax.ShapeDtypeStruct((NC, N_NODES, N_FEATS), jnp.float32),
        mesh=_MESH,
        scratch_types=[
            pltpu.VMEM((2, IDXG, CHUNK), jnp.int32),
            pltpu.VMEM((2, IDXG, CHUNK), jnp.int32),
            pltpu.VMEM((NBUF, CHUNK, N_FEATS), jnp.float32),
            pltpu.VMEM_SHARED((N_NODES, N_FEATS), jnp.float32),
            pltpu.SemaphoreType.DMA,
            pltpu.SemaphoreType.DMA,
            pltpu.SemaphoreType.DMA,
            pltpu.SemaphoreType.DMA,
            pltpu.SemaphoreType.DMA,
            pltpu.SemaphoreType.DMA,
            pltpu.SemaphoreType.DMA,
            pltpu.SemaphoreType.DMA,
            pltpu.SemaphoreType.DMA,
            pltpu.SemaphoreType.DMA,
        ],
        compiler_params=_SC_PARAMS,
    )(h, e3d, zeros)


def _finalize_body(p_ref, n2_ref, b_ref, out_ref):
    out_ref[...] = (p_ref[0] + p_ref[1]) * n2_ref[...] + b_ref[...]


def _finalize(partials, norm_dst, bias):
    return pl.pallas_call(
        _finalize_body,
        grid=(N_NODES // _MM_BLK,),
        in_specs=[
            pl.BlockSpec((NC, _MM_BLK, N_FEATS), lambda i: (0, i, 0)),
            pl.BlockSpec((_MM_BLK, 1), lambda i: (i, 0)),
            pl.BlockSpec((1, N_FEATS), lambda i: (0, 0)),
        ],
        out_specs=pl.BlockSpec((_MM_BLK, N_FEATS), lambda i: (i, 0)),
        out_shape=jax.ShapeDtypeStruct((N_NODES, N_FEATS), jnp.float32),
    )(partials, norm_dst, bias)


def kernel(features, edge_index, weight, bias):
    ei = edge_index.astype(jnp.int32)
    e3d = ei.reshape(2, NW * CHUNKS_PER_TILE, CHUNK)
    hsrc, hdst = _degrees(e3d)
    norm_src, norm_dst = _norms(hsrc, hdst)
    h = _matmul(features, norm_src, weight)
    zeros = jnp.zeros((CP_ROWS, N_FEATS), jnp.float32)
    partials = _message(h, e3d, zeros)
    return _finalize(partials, norm_dst, bias.reshape(1, N_FEATS))


# speedup vs baseline: 1.0731x; 1.0731x over previous
import functools

import jax
import jax.numpy as jnp
from jax import lax
from jax.experimental import pallas as pl
from jax.experimental.pallas import tpu as pltpu
from jax.experimental.pallas import tpu_sc as plsc

N_NODES = 10000
N_FEATS = 128
N_EDGES = 320000

NC = 2
NS = 16
NW = NC * NS
EDGES_PER_TILE = N_EDGES // NW
CHUNK = 50
CHUNKS_PER_TILE = EDGES_PER_TILE // CHUNK
IDXG = 40
N_GROUPS = CHUNKS_PER_TILE // IDXG
INNER = 8
NBUF = 4
LOOKAHEAD = 3
LANES = 16
CP_ROWS = 200
CP_CHUNKS = N_NODES // CP_ROWS
CP_ROUNDS = -(-CP_CHUNKS // NS)

_MESH = plsc.VectorSubcoreMesh(
    core_axis_name="c", subcore_axis_name="s", num_cores=NC, num_subcores=NS)
_SC_PARAMS = pltpu.CompilerParams(needs_layout_passes=False)


HIST_ROWS = 80


def _degrees_body(edges_hbm, out_src, out_dst,
                  idx_v, hist2d, idn_v, hsh_src, hsh_dst):
    c = lax.axis_index("c")
    s = lax.axis_index("s")
    wid = c * NS + s
    base = wid * EDGES_PER_TILE
    ones = jnp.full((LANES,), 1.0, jnp.float32)
    zeros = jnp.zeros((LANES,), jnp.float32)

    def zero_hist(i, _):
        for k in range(N_FEATS // LANES):
            hist2d[i, pl.ds(k * LANES, LANES)] = zeros
        return 0

    def fill_idn(i, _):
        idn_v[pl.ds(i * LANES, LANES)] = lax.iota(jnp.int32, LANES) + i * LANES
        return 0

    lax.fori_loop(0, HIST_ROWS, zero_hist, 0)
    lax.fori_loop(0, HIST_ROWS // LANES, fill_idn, 0)

    @pl.when(s == 0)
    def _():
        pltpu.sync_copy(hist2d, hsh_src)
        pltpu.sync_copy(hist2d, hsh_dst)

    plsc.subcore_barrier()

    def one_endpoint(ep_base, hist_sh, out_hbm, rezero):
        pltpu.sync_copy(edges_hbm.at[pl.ds(ep_base + base, EDGES_PER_TILE)],
                        idx_v)

        def add_step(i, _):
            idx = idx_v[pl.ds(i * LANES, LANES)]
            plsc.addupdate_scatter(hist2d, [idx >> 7, idx & 127], ones)
            return 0

        lax.fori_loop(0, EDGES_PER_TILE // LANES, add_step, 0)
        pltpu.sync_copy(hist2d, hist_sh.at[idn_v], add=True)
        if rezero:
            lax.fori_loop(0, HIST_ROWS, zero_hist, 0)
        plsc.subcore_barrier()

        @pl.when(s == 0)
        def _():
            pltpu.sync_copy(hist_sh, out_hbm.at[c])

    one_endpoint(0, hsh_src, out_src, True)
    one_endpoint(N_EDGES, hsh_dst, out_dst, False)


def _degrees(edges_flat):
    return pl.kernel(
        _degrees_body,
        out_type=(
            jax.ShapeDtypeStruct((NC, HIST_ROWS, N_FEATS), jnp.float32),
            jax.ShapeDtypeStruct((NC, HIST_ROWS, N_FEATS), jnp.float32),
        ),
        mesh=_MESH,
        scratch_types=[
            pltpu.VMEM((EDGES_PER_TILE,), jnp.int32),
            pltpu.VMEM((HIST_ROWS, N_FEATS), jnp.float32),
            pltpu.VMEM((HIST_ROWS,), jnp.int32),
            pltpu.VMEM_SHARED((HIST_ROWS, N_FEATS), jnp.float32),
            pltpu.VMEM_SHARED((HIST_ROWS, N_FEATS), jnp.float32),
        ],
        compiler_params=_SC_PARAMS,
    )(edges_flat)


_MM_BLK = 1000


def _norms_body(hs_ref, hd_ref, ns_ref, nd_ref):
    def norm(h2):
        deg = (h2[0] + h2[1]).reshape(HIST_ROWS * N_FEATS)[:N_NODES]
        return lax.rsqrt(jnp.maximum(deg, 1.0))[:, None]

    ns_ref[...] = norm(hs_ref[...])
    nd_ref[...] = norm(hd_ref[...])


def _norms(hist_src, hist_dst):
    return pl.pallas_call(
        _norms_body,
        out_shape=(
            jax.ShapeDtypeStruct((N_NODES, 1), jnp.float32),
            jax.ShapeDtypeStruct((N_NODES, 1), jnp.float32),
        ),
    )(hist_src, hist_dst)


def _matmul_body(x_ref, n1_ref, w_ref, h_ref):
    h_ref[...] = jnp.dot(x_ref[...] * n1_ref[...], w_ref[...],
                         preferred_element_type=jnp.float32)


def _matmul(features, norm_src, weight):
    return pl.pallas_call(
        _matmul_body,
        grid=(N_NODES // _MM_BLK,),
        in_specs=[
            pl.BlockSpec((_MM_BLK, N_FEATS), lambda i: (i, 0)),
            pl.BlockSpec((_MM_BLK, 1), lambda i: (i, 0)),
            pl.BlockSpec((N_FEATS, N_FEATS), lambda i: (0, 0)),
        ],
        out_specs=pl.BlockSpec((_MM_BLK, N_FEATS), lambda i: (i, 0)),
        out_shape=jax.ShapeDtypeStruct((N_NODES, N_FEATS), jnp.float32),
    )(features, norm_src, weight)


def _message_body(h_hbm, e3d_hbm, zeros_hbm, out_hbm,
                  srcblk, dstblk, rows, acc_sh,
                  gsem0, gsem1, gsem2, gsem3, ssem0, ssem1, ssem2, ssem3,
                  isrc, idst):
    c = lax.axis_index("c")
    s = lax.axis_index("s")
    wid = c * NS + s

    for j in range(CP_ROUNDS):
        cid = s + NS * j

        @pl.when(cid < CP_CHUNKS)
        def _():
            pltpu.sync_copy(zeros_hbm, acc_sh.at[pl.ds(cid * CP_ROWS, CP_ROWS)])

    gsems = (gsem0, gsem1, gsem2, gsem3)
    ssems = (ssem0, ssem1, ssem2, ssem3)
    base = wid * CHUNKS_PER_TILE

    def gather(slot, j, b):
        return pltpu.async_copy(h_hbm.at[srcblk.at[slot, j]], rows.at[b],
                                gsems[b])

    def wait_gather(b):
        pltpu.make_async_copy(h_hbm.at[srcblk.at[0, 0]], rows.at[b],
                              gsems[b]).wait()

    def scatter(slot, j, b):
        return pltpu.async_copy(rows.at[b], acc_sh.at[dstblk.at[slot, j]],
                                ssems[b], add=True)

    def wait_scatter(b):
        pltpu.make_async_copy(rows.at[b], acc_sh.at[dstblk.at[0, 0]],
                              ssems[b]).wait()

    def load_idx(g1, slot, sem_pair):
        nbase = base + g1 * IDXG
        pltpu.async_copy(e3d_hbm.at[0, pl.ds(nbase, IDXG)],
                         srcblk.at[slot], sem_pair[0])
        pltpu.async_copy(e3d_hbm.at[1, pl.ds(nbase, IDXG)],
                         dstblk.at[slot], sem_pair[1])

    def wait_idx(slot):
        pltpu.make_async_copy(e3d_hbm.at[0, pl.ds(base, IDXG)],
                              srcblk.at[slot], isrc).wait()
        pltpu.make_async_copy(e3d_hbm.at[1, pl.ds(base, IDXG)],
                              dstblk.at[slot], idst).wait()

    pltpu.sync_copy(e3d_hbm.at[0, pl.ds(base, IDXG)], srcblk.at[0])
    pltpu.sync_copy(e3d_hbm.at[1, pl.ds(base, IDXG)], dstblk.at[0])
    for p in range(LOOKAHEAD):
        gather(0, p, p % NBUF)
    plsc.subcore_barrier()

    def group_step(g, _):
        slot = g % 2
        nslot = (g + 1) % 2

        def inner_step(q, _):
            for k in range(INNER):
                j = q * INNER + k
                jn = j + LOOKAHEAD
                b = k % NBUF
                bn = (k + LOOKAHEAD) % NBUF
                pb = (k - 1) % NBUF
                if k == 0:
                    @pl.when((g > 0) | (q > 0))
                    def _():
                        wait_scatter(pb)

                    @pl.when((q == 0) & (g + 1 < N_GROUPS))
                    def _():
                        load_idx(g + 1, nslot, (isrc, idst))
                else:
                    wait_scatter(pb)

                if k + LOOKAHEAD < INNER:
                    gather(slot, jn, bn)
                else:
                    @pl.when(jn < IDXG)
                    def _():
                        gather(slot, jn, bn)

                    @pl.when((jn >= IDXG) & (g + 1 < N_GROUPS))
                    def _():
                        if k == INNER - LOOKAHEAD:
                            wait_idx(nslot)
                        gather(nslot, jn - IDXG, bn)

                wait_gather(b)
                scatter(slot, j, b)
            return 0

        lax.fori_loop(0, IDXG // INNER, inner_step, 0)
        return 0

    lax.fori_loop(0, N_GROUPS, group_step, 0)
    wait_scatter((IDXG - 1) % NBUF)
    plsc.subcore_barrier()

    for j in range(CP_ROUNDS):
        cid = s + NS * j

        @pl.when(cid < CP_CHUNKS)
        def _():
            pltpu.sync_copy(acc_sh.at[pl.ds(cid * CP_ROWS, CP_ROWS)],
                            out_hbm.at[c, pl.ds(cid * CP_ROWS, CP_ROWS)])


def _message(h, e3d, zeros):
    return pl.kernel(
        _message_body,
        out_type=jax.ShapeDtypeStruct((NC, N_NODES, N_FEATS), jnp.float32),
        mesh=_MESH,
        scratch_types=[
            pltpu.VMEM((2, IDXG, CHUNK), jnp.int32),
            pltpu.VMEM((2, IDXG, CHUNK), jnp.int32),
            pltpu.VMEM((NBUF, CHUNK, N_FEATS), jnp.float32),
            pltpu.VMEM_SHARED((N_NODES, N_FEATS), jnp.float32),
            pltpu.SemaphoreType.DMA,
            pltpu.SemaphoreType.DMA,
            pltpu.SemaphoreType.DMA,
            pltpu.SemaphoreType.DMA,
            pltpu.SemaphoreType.DMA,
            pltpu.SemaphoreType.DMA,
            pltpu.SemaphoreType.DMA,
            pltpu.SemaphoreType.DMA,
            pltpu.SemaphoreType.DMA,
            pltpu.SemaphoreType.DMA,
        ],
        compiler_params=_SC_PARAMS,
    )(h, e3d, zeros)


def _finalize_body(p_ref, n2_ref, b_ref, out_ref):
    out_ref[...] = (p_ref[0] + p_ref[1]) * n2_ref[...] + b_ref[...]


def _finalize(partials, norm_dst, bias):
    return pl.pallas_call(
        _finalize_body,
        grid=(N_NODES // _MM_BLK,),
        in_specs=[
            pl.BlockSpec((NC, _MM_BLK, N_FEATS), lambda i: (0, i, 0)),
            pl.BlockSpec((_MM_BLK, 1), lambda i: (i, 0)),
            pl.BlockSpec((1, N_FEATS), lambda i: (0, 0)),
        ],
        out_specs=pl.BlockSpec((_MM_BLK, N_FEATS), lambda i: (i, 0)),
        out_shape=jax.ShapeDtypeStruct((N_NODES, N_FEATS), jnp.float32),
    )(partials, norm_dst, bias)


def kernel(features, edge_index, weight, bias):
    ei = edge_index.astype(jnp.int32)
    edges_flat = ei.reshape(2 * N_EDGES)
    e3d = ei.reshape(2, NW * CHUNKS_PER_TILE, CHUNK)
    hsrc, hdst = _degrees(edges_flat)
    norm_src, norm_dst = _norms(hsrc, hdst)
    h = _matmul(features, norm_src, weight)
    zeros = jnp.zeros((CP_ROWS, N_FEATS), jnp.float32)
    partials = _message(h, e3d, zeros)
    return _finalize(partials, norm_dst, bias.reshape(1, N_FEATS))
